# baseline (speedup 1.0000x reference)
import jax
import jax.numpy as jnp
from jax.experimental import pallas as pl
from jax.experimental.pallas import tpu as pltpu

_ISSUE_UNROLL = 32
_TILE = 1024


def _fused_kernel(Dw, Dp, ids_smem, w_hbm, pos_hbm, out_ref, sem_w, sem_w2, sem_p):
    tile = out_ref.shape[0]
    base = pl.program_id(0) * tile

    pcopy = pltpu.make_async_copy(
        pos_hbm.at[pl.ds(base, tile)],
        out_ref.at[:, pl.ds(Dw, Dp)],
        sem_p,
    )
    pcopy.start()

    unroll = _ISSUE_UNROLL if tile % _ISSUE_UNROLL == 0 else 1

    @pl.loop(0, tile // unroll)
    def _issue(c):
        r0 = c * unroll
        for u in range(unroll):
            r = r0 + u
            tok = ids_smem[base + r]
            pltpu.make_async_copy(
                w_hbm.at[tok],
                out_ref.at[r, pl.ds(0, Dw)],
                sem_w if u % 2 == 0 else sem_w2,
            ).start()

    half = tile // 2
    pltpu.make_async_copy(
        w_hbm.at[pl.ds(0, half)],
        out_ref.at[pl.ds(0, half), pl.ds(0, Dw)],
        sem_w,
    ).wait()
    pltpu.make_async_copy(
        w_hbm.at[pl.ds(0, half)],
        out_ref.at[pl.ds(0, half), pl.ds(0, Dw)],
        sem_w2,
    ).wait()
    pcopy.wait()


def _pick_tile(L):
    if L <= _TILE:
        return L
    for t in (_TILE, 512, 256, 128, 64, 32, 16, 8):
        if L % t == 0:
            return t
    return L


def kernel(ids, word_table, pos_table):
    L = ids.shape[0]
    V, Dw = word_table.shape
    P, Dp = pos_table.shape
    assert P >= L, "position table must cover the sequence length"

    ids = ids.astype(jnp.int32)
    pos_used = pos_table[:L]
    tile = _pick_tile(L)
    grid = (L // tile,)

    out = pl.pallas_call(
        lambda *refs: _fused_kernel(Dw, Dp, *refs),
        out_shape=jax.ShapeDtypeStruct((L, Dw + Dp), word_table.dtype),
        grid_spec=pltpu.PrefetchScalarGridSpec(
            num_scalar_prefetch=1,
            grid=grid,
            in_specs=[
                pl.BlockSpec(memory_space=pl.ANY),
                pl.BlockSpec(memory_space=pl.ANY),
            ],
            out_specs=pl.BlockSpec((tile, Dw + Dp), lambda i, ids_ref: (i, 0)),
            scratch_shapes=[pltpu.SemaphoreType.DMA(()),
                            pltpu.SemaphoreType.DMA(()),
                            pltpu.SemaphoreType.DMA(())],
        ),
        compiler_params=pltpu.CompilerParams(
            dimension_semantics=("parallel",),
            disable_bounds_checks=True,
        ),
    )(ids, word_table, pos_used)
    return out
